# initial kernel scaffold (unmeasured)
import jax
import jax.numpy as jnp
from jax import lax
from jax.experimental import pallas as pl
from jax.experimental.pallas import tpu as pltpu

T = 1024
TL = 512
D = 1024
F = 2048
FC = 1024
E = 8
EL = 4


def _peer():
    return (lax.axis_index("x"), lax.axis_index("y"), 1 - lax.axis_index("z"))


def _exchange_body(x_ref, rt_ref, xf_ref, rtf_ref, send_sems, recv_sems):
    my_z = lax.axis_index("z")
    peer = _peer()

    xf_ref[pl.ds(my_z * TL, TL), :] = x_ref[...]
    rtf_ref[pl.ds(my_z * EL, EL), :] = rt_ref[...]

    rdma_x = pltpu.make_async_remote_copy(
        src_ref=x_ref,
        dst_ref=xf_ref.at[pl.ds(my_z * TL, TL), :],
        send_sem=send_sems.at[0],
        recv_sem=recv_sems.at[0],
        device_id=peer,
        device_id_type=pl.DeviceIdType.MESH,
    )
    rdma_r = pltpu.make_async_remote_copy(
        src_ref=rt_ref,
        dst_ref=rtf_ref.at[pl.ds(my_z * EL, EL), :],
        send_sem=send_sems.at[1],
        recv_sem=recv_sems.at[1],
        device_id=peer,
        device_id_type=pl.DeviceIdType.MESH,
    )
    rdma_x.start()
    rdma_r.start()
    rdma_x.wait()
    rdma_r.wait()


def _exchange(x, router_t):
    return pl.pallas_call(
        _exchange_body,
        out_shape=(
            jax.ShapeDtypeStruct((T, D), jnp.float32),
            jax.ShapeDtypeStruct((E, D), jnp.float32),
        ),
        in_specs=[
            pl.BlockSpec(memory_space=pltpu.VMEM),
            pl.BlockSpec(memory_space=pltpu.VMEM),
        ],
        out_specs=(
            pl.BlockSpec(memory_space=pltpu.VMEM),
            pl.BlockSpec(memory_space=pltpu.VMEM),
        ),
        scratch_shapes=[
            pltpu.SemaphoreType.DMA((2,)),
            pltpu.SemaphoreType.DMA((2,)),
        ],
        compiler_params=pltpu.CompilerParams(collective_id=0),
    )(x, router_t)


def _ffn_body(xf_ref, gw_ref, w1_ref, w2_ref, out_ref):
    e = pl.program_id(0)
    f = pl.program_id(1)
    h = jnp.maximum(
        jnp.dot(xf_ref[...], w1_ref[0], preferred_element_type=jnp.float32), 0.0
    )
    contrib = jnp.dot(h * gw_ref[0], w2_ref[0], preferred_element_type=jnp.float32)

    @pl.when(jnp.logical_and(e == 0, f == 0))
    def _():
        out_ref[...] = contrib

    @pl.when(jnp.logical_not(jnp.logical_and(e == 0, f == 0)))
    def _():
        out_ref[...] = out_ref[...] + contrib


def _ffn(x_full, gw3, W1, W2):
    return pl.pallas_call(
        _ffn_body,
        grid=(EL, F // FC),
        out_shape=jax.ShapeDtypeStruct((T, D), jnp.float32),
        in_specs=[
            pl.BlockSpec((T, D), lambda e, f: (0, 0)),
            pl.BlockSpec((1, T, 1), lambda e, f: (e, 0, 0)),
            pl.BlockSpec((1, D, FC), lambda e, f: (e, 0, f)),
            pl.BlockSpec((1, FC, D), lambda e, f: (e, f, 0)),
        ],
        out_specs=pl.BlockSpec((T, D), lambda e, f: (0, 0)),
    )(x_full, gw3, W1, W2)


def _reduce_body(p_ref, out_ref, recv_ref, send_sem, recv_sem):
    my_z = lax.axis_index("z")
    peer = _peer()

    rdma = pltpu.make_async_remote_copy(
        src_ref=p_ref.at[pl.ds((1 - my_z) * TL, TL), :],
        dst_ref=recv_ref,
        send_sem=send_sem,
        recv_sem=recv_sem,
        device_id=peer,
        device_id_type=pl.DeviceIdType.MESH,
    )
    rdma.start()
    rdma.wait()
    out_ref[...] = p_ref[pl.ds(my_z * TL, TL), :] + recv_ref[...]


def _reduce(partial):
    return pl.pallas_call(
        _reduce_body,
        out_shape=jax.ShapeDtypeStruct((TL, D), jnp.float32),
        in_specs=[pl.BlockSpec(memory_space=pltpu.VMEM)],
        out_specs=pl.BlockSpec(memory_space=pltpu.VMEM),
        scratch_shapes=[
            pltpu.VMEM((TL, D), jnp.float32),
            pltpu.SemaphoreType.DMA,
            pltpu.SemaphoreType.DMA,
        ],
        compiler_params=pltpu.CompilerParams(collective_id=1),
    )(partial)


def kernel(x, router, W1, W2):
    my_z = lax.axis_index("z")

    x_full, router_t_full = _exchange(x, router.T)

    gates = x_full @ router_t_full.T
    i1 = jnp.argmax(gates, axis=1)
    v1 = jnp.max(gates, axis=1)
    masked = gates - jnp.where(
        jax.nn.one_hot(i1, E, dtype=jnp.float32) > 0, jnp.inf, 0.0
    )
    i2 = jnp.argmax(masked, axis=1)
    v2 = jnp.max(masked, axis=1)
    e2 = jnp.exp(v2 - v1)
    w1 = 1.0 / (1.0 + e2)
    w2 = e2 / (1.0 + e2)
    gw = (
        w1[:, None] * jax.nn.one_hot(i1, E, dtype=jnp.float32)
        + w2[:, None] * jax.nn.one_hot(i2, E, dtype=jnp.float32)
    )
    gw_local = lax.dynamic_slice(gw, (0, my_z * EL), (T, EL))
    gw3 = gw_local.T[:, :, None]

    partial = _ffn(x_full, gw3, W1, W2)

    return _reduce(partial)


# baseline (device time: 112670 ns/iter reference)
import jax
import jax.numpy as jnp
from jax import lax
from jax.experimental import pallas as pl
from jax.experimental.pallas import tpu as pltpu

T = 1024
TL = 512
D = 1024
F = 2048
FC = 1024
E = 8
EL = 4


def _peer():
    return (lax.axis_index("x"), lax.axis_index("y"), 1 - lax.axis_index("z"))


def _exchange_body(x_ref, rt_ref, xf_ref, rtf_ref, send_sems, recv_sems):
    my_z = lax.axis_index("z")
    peer = _peer()

    xf_ref[pl.ds(my_z * TL, TL), :] = x_ref[...]
    rtf_ref[pl.ds(my_z * 8, 8), :] = rt_ref[...]

    rdma_x = pltpu.make_async_remote_copy(
        src_ref=x_ref,
        dst_ref=xf_ref.at[pl.ds(my_z * TL, TL), :],
        send_sem=send_sems.at[0],
        recv_sem=recv_sems.at[0],
        device_id=peer,
        device_id_type=pl.DeviceIdType.MESH,
    )
    rdma_r = pltpu.make_async_remote_copy(
        src_ref=rt_ref,
        dst_ref=rtf_ref.at[pl.ds(my_z * 8, 8), :],
        send_sem=send_sems.at[1],
        recv_sem=recv_sems.at[1],
        device_id=peer,
        device_id_type=pl.DeviceIdType.MESH,
    )
    rdma_x.start()
    rdma_r.start()
    rdma_x.wait()
    rdma_r.wait()


def _exchange(x, router_t):
    return pl.pallas_call(
        _exchange_body,
        out_shape=(
            jax.ShapeDtypeStruct((T, D), jnp.float32),
            jax.ShapeDtypeStruct((16, D), jnp.float32),
        ),
        in_specs=[
            pl.BlockSpec(memory_space=pltpu.VMEM),
            pl.BlockSpec(memory_space=pltpu.VMEM),
        ],
        out_specs=(
            pl.BlockSpec(memory_space=pltpu.VMEM),
            pl.BlockSpec(memory_space=pltpu.VMEM),
        ),
        scratch_shapes=[
            pltpu.SemaphoreType.DMA((2,)),
            pltpu.SemaphoreType.DMA((2,)),
        ],
    )(x, router_t)


def _ffn_body(xf_ref, gw_ref, w1_ref, w2_ref, out_ref):
    e = pl.program_id(0)
    f = pl.program_id(1)
    h = jnp.maximum(
        jnp.dot(xf_ref[...], w1_ref[0], preferred_element_type=jnp.float32), 0.0
    )
    contrib = jnp.dot(h * gw_ref[0], w2_ref[0], preferred_element_type=jnp.float32)

    @pl.when(jnp.logical_and(e == 0, f == 0))
    def _():
        out_ref[...] = contrib

    @pl.when(jnp.logical_not(jnp.logical_and(e == 0, f == 0)))
    def _():
        out_ref[...] = out_ref[...] + contrib


def _ffn(x_full, gw3, W1, W2):
    return pl.pallas_call(
        _ffn_body,
        grid=(EL, F // FC),
        out_shape=jax.ShapeDtypeStruct((T, D), jnp.float32),
        in_specs=[
            pl.BlockSpec((T, D), lambda e, f: (0, 0)),
            pl.BlockSpec((1, T, 1), lambda e, f: (e, 0, 0)),
            pl.BlockSpec((1, D, FC), lambda e, f: (e, 0, f)),
            pl.BlockSpec((1, FC, D), lambda e, f: (e, f, 0)),
        ],
        out_specs=pl.BlockSpec((T, D), lambda e, f: (0, 0)),
    )(x_full, gw3, W1, W2)


def _reduce_body(p_ref, out_ref, recv_ref, send_sem, recv_sem):
    my_z = lax.axis_index("z")
    peer = _peer()

    rdma = pltpu.make_async_remote_copy(
        src_ref=p_ref.at[pl.ds((1 - my_z) * TL, TL), :],
        dst_ref=recv_ref,
        send_sem=send_sem,
        recv_sem=recv_sem,
        device_id=peer,
        device_id_type=pl.DeviceIdType.MESH,
    )
    rdma.start()
    rdma.wait()
    out_ref[...] = p_ref[pl.ds(my_z * TL, TL), :] + recv_ref[...]


def _reduce(partial):
    return pl.pallas_call(
        _reduce_body,
        out_shape=jax.ShapeDtypeStruct((TL, D), jnp.float32),
        in_specs=[pl.BlockSpec(memory_space=pltpu.VMEM)],
        out_specs=pl.BlockSpec(memory_space=pltpu.VMEM),
        scratch_shapes=[
            pltpu.VMEM((TL, D), jnp.float32),
            pltpu.SemaphoreType.DMA,
            pltpu.SemaphoreType.DMA,
        ],
    )(partial)


def kernel(x, router, W1, W2):
    my_z = lax.axis_index("z")

    rt_pad = jnp.zeros((8, D), jnp.float32).at[:EL].set(router.T)
    x_full, rtf_pad = _exchange(x, rt_pad)
    router_t_full = jnp.concatenate([rtf_pad[:EL], rtf_pad[8 : 8 + EL]], axis=0)

    gates = jnp.dot(
        x_full, router_t_full.T, precision=lax.Precision.HIGHEST
    )
    i1 = jnp.argmax(gates, axis=1)
    v1 = jnp.max(gates, axis=1)
    masked = gates - jnp.where(
        jax.nn.one_hot(i1, E, dtype=jnp.float32) > 0, jnp.inf, 0.0
    )
    i2 = jnp.argmax(masked, axis=1)
    v2 = jnp.max(masked, axis=1)
    e2 = jnp.exp(v2 - v1)
    w1 = 1.0 / (1.0 + e2)
    w2 = e2 / (1.0 + e2)
    gw = (
        w1[:, None] * jax.nn.one_hot(i1, E, dtype=jnp.float32)
        + w2[:, None] * jax.nn.one_hot(i2, E, dtype=jnp.float32)
    )
    gw_local = lax.dynamic_slice(gw, (0, my_z * EL), (T, EL))
    gw3 = gw_local.T[:, :, None]

    partial = _ffn(x_full, gw3, W1, W2)

    return _reduce(partial)


# device time: 78695 ns/iter; 1.4317x vs baseline; 1.4317x over previous
import jax
import jax.numpy as jnp
from jax import lax
from jax.experimental import pallas as pl
from jax.experimental.pallas import tpu as pltpu

T = 1024
TL = 512
D = 1024
F = 2048
FC = 1024
NF = F // FC
E = 8
EL = 4

MESH = pl.DeviceIdType.MESH


def _peer():
    return (lax.axis_index("x"), lax.axis_index("y"), 1 - lax.axis_index("z"))


def _router_body(rt_ref, rtf_ref, send_sem, recv_sem):
    my_z = lax.axis_index("z")
    peer = _peer()

    barrier = pltpu.get_barrier_semaphore()
    pl.semaphore_signal(barrier, inc=1, device_id=peer, device_id_type=MESH)
    pl.semaphore_wait(barrier, 1)

    rtf_ref[pl.ds(my_z * 8, 8), :] = rt_ref[...]
    rdma = pltpu.make_async_remote_copy(
        src_ref=rt_ref,
        dst_ref=rtf_ref.at[pl.ds(my_z * 8, 8), :],
        send_sem=send_sem,
        recv_sem=recv_sem,
        device_id=peer,
        device_id_type=MESH,
    )
    rdma.start()
    rdma.wait()


def _router_exchange(rt_pad):
    return pl.pallas_call(
        _router_body,
        out_shape=jax.ShapeDtypeStruct((16, D), jnp.float32),
        in_specs=[pl.BlockSpec(memory_space=pltpu.VMEM)],
        out_specs=pl.BlockSpec(memory_space=pltpu.VMEM),
        scratch_shapes=[pltpu.SemaphoreType.DMA, pltpu.SemaphoreType.DMA],
        compiler_params=pltpu.CompilerParams(collective_id=0),
    )(rt_pad)


def _mega_body(
    xbf_ref, gws_ref, gwm_ref, w1_ref, w2_ref, out_ref,
    xall_ref, gwall_ref, pall_ref, psend_ref, precv_ref, send_sems, recv_sems,
):
    h = pl.program_id(0)
    e = pl.program_id(1)
    f = pl.program_id(2)
    my_z = lax.axis_index("z")
    peer = _peer()
    my_off = my_z * TL
    peer_off = (1 - my_z) * TL

    def xfer_x():
        return pltpu.make_async_remote_copy(
            src_ref=xbf_ref,
            dst_ref=xall_ref.at[pl.ds(my_off, TL), :],
            send_sem=send_sems.at[0],
            recv_sem=recv_sems.at[0],
            device_id=peer,
            device_id_type=MESH,
        )

    def xfer_gw():
        return pltpu.make_async_remote_copy(
            src_ref=gws_ref,
            dst_ref=gwall_ref.at[:, pl.ds(my_off, TL), :],
            send_sem=send_sems.at[1],
            recv_sem=recv_sems.at[1],
            device_id=peer,
            device_id_type=MESH,
        )

    def xfer_p():
        return pltpu.make_async_remote_copy(
            src_ref=psend_ref,
            dst_ref=precv_ref,
            send_sem=send_sems.at[2],
            recv_sem=recv_sems.at[2],
            device_id=peer,
            device_id_type=MESH,
        )

    @pl.when(jnp.logical_and(h == 0, jnp.logical_and(e == 0, f == 0)))
    def _():
        barrier = pltpu.get_barrier_semaphore()
        pl.semaphore_signal(barrier, inc=1, device_id=peer, device_id_type=MESH)
        pl.semaphore_wait(barrier, 1)
        xall_ref[pl.ds(my_off, TL), :] = xbf_ref[...]
        gwall_ref[:, pl.ds(my_off, TL), :] = gwm_ref[...]
        xfer_x().start()
        xfer_gw().start()

    @pl.when(jnp.logical_and(h == 1, jnp.logical_and(e == 0, f == 0)))
    def _():
        xfer_x().wait_recv()
        xfer_gw().wait_recv()

    off = jnp.where(h == 0, my_off, peer_off)
    xh = xall_ref[pl.ds(off, TL), :]
    hid = jnp.maximum(
        jnp.dot(xh, w1_ref[0].astype(jnp.bfloat16),
                preferred_element_type=jnp.float32),
        0.0,
    )
    hid = (hid * gwall_ref[e, pl.ds(off, TL), :]).astype(jnp.bfloat16)
    contrib = jnp.dot(hid, w2_ref[0].astype(jnp.bfloat16),
                      preferred_element_type=jnp.float32)

    @pl.when(jnp.logical_and(e == 0, f == 0))
    def _():
        pall_ref[pl.ds(off, TL), :] = contrib

    @pl.when(jnp.logical_not(jnp.logical_and(e == 0, f == 0)))
    def _():
        pall_ref[pl.ds(off, TL), :] = pall_ref[pl.ds(off, TL), :] + contrib

    @pl.when(jnp.logical_and(h == 1, jnp.logical_and(e == EL - 1, f == NF - 1)))
    def _():
        psend_ref[...] = pall_ref[pl.ds(peer_off, TL), :].astype(jnp.bfloat16)
        p = xfer_p()
        p.start()
        p.wait_recv()
        out_ref[...] = pall_ref[pl.ds(my_off, TL), :] + precv_ref[...].astype(
            jnp.float32
        )
        xfer_x().wait_send()
        xfer_gw().wait_send()
        p.wait_send()


def _mega(x_bf, gw_send, gw_mine, W1, W2):
    grid = (2, EL, NF)
    return pl.pallas_call(
        _mega_body,
        grid=grid,
        out_shape=jax.ShapeDtypeStruct((TL, D), jnp.float32),
        in_specs=[
            pl.BlockSpec(memory_space=pltpu.VMEM),
            pl.BlockSpec(memory_space=pltpu.VMEM),
            pl.BlockSpec(memory_space=pltpu.VMEM),
            pl.BlockSpec((1, D, FC), lambda h, e, f: (e, 0, f)),
            pl.BlockSpec((1, FC, D), lambda h, e, f: (e, f, 0)),
        ],
        out_specs=pl.BlockSpec((TL, D), lambda h, e, f: (0, 0)),
        scratch_shapes=[
            pltpu.VMEM((T, D), jnp.bfloat16),
            pltpu.VMEM((EL, T, 1), jnp.float32),
            pltpu.VMEM((T, D), jnp.float32),
            pltpu.VMEM((TL, D), jnp.bfloat16),
            pltpu.VMEM((TL, D), jnp.bfloat16),
            pltpu.SemaphoreType.DMA((3,)),
            pltpu.SemaphoreType.DMA((3,)),
        ],
        compiler_params=pltpu.CompilerParams(collective_id=1),
    )(x_bf, gw_send, gw_mine, W1, W2)


def kernel(x, router, W1, W2):
    my_z = lax.axis_index("z")

    rt_pad = jnp.zeros((8, D), jnp.float32).at[:EL].set(router.T)
    rtf_pad = _router_exchange(rt_pad)
    router_t_full = jnp.concatenate([rtf_pad[:EL], rtf_pad[8 : 8 + EL]], axis=0)

    gates = jnp.dot(x, router_t_full.T, precision=lax.Precision.HIGHEST)
    i1 = jnp.argmax(gates, axis=1)
    v1 = jnp.max(gates, axis=1)
    masked = gates - jnp.where(
        jax.nn.one_hot(i1, E, dtype=jnp.float32) > 0, jnp.inf, 0.0
    )
    i2 = jnp.argmax(masked, axis=1)
    v2 = jnp.max(masked, axis=1)
    e2 = jnp.exp(v2 - v1)
    w1g = 1.0 / (1.0 + e2)
    w2g = e2 / (1.0 + e2)
    gw = (
        w1g[:, None] * jax.nn.one_hot(i1, E, dtype=jnp.float32)
        + w2g[:, None] * jax.nn.one_hot(i2, E, dtype=jnp.float32)
    )
    gw_mine = lax.dynamic_slice(gw, (0, my_z * EL), (TL, EL)).T[:, :, None]
    gw_send = lax.dynamic_slice(gw, (0, (1 - my_z) * EL), (TL, EL)).T[:, :, None]

    return _mega(x.astype(jnp.bfloat16), gw_send, gw_mine, W1, W2)


# device time: 65930 ns/iter; 1.7089x vs baseline; 1.1936x over previous
import jax
import jax.numpy as jnp
from jax import lax
from jax.experimental import pallas as pl
from jax.experimental.pallas import tpu as pltpu

T = 1024
TL = 512
TC = 128
D = 1024
F = 2048
E = 8
EL = 4

MESH = pl.DeviceIdType.MESH


def _zpeer():
    return (lax.axis_index("x"), lax.axis_index("y"), 1 - lax.axis_index("z"))


def _router_body(rt_ref, rtf_ref, send_sem, recv_sem):
    my_z = lax.axis_index("z")
    peer = _zpeer()

    barrier = pltpu.get_barrier_semaphore()
    pl.semaphore_signal(barrier, inc=1, device_id=peer, device_id_type=MESH)
    pl.semaphore_wait(barrier, 1)

    rtf_ref[pl.ds(my_z * 8, 8), :] = rt_ref[...]
    rdma = pltpu.make_async_remote_copy(
        src_ref=rt_ref,
        dst_ref=rtf_ref.at[pl.ds(my_z * 8, 8), :],
        send_sem=send_sem,
        recv_sem=recv_sem,
        device_id=peer,
        device_id_type=MESH,
    )
    rdma.start()
    rdma.wait()


def _router_exchange(rt_pad):
    return pl.pallas_call(
        _router_body,
        out_shape=jax.ShapeDtypeStruct((16, D), jnp.float32),
        in_specs=[pl.BlockSpec(memory_space=pltpu.VMEM)],
        out_specs=pl.BlockSpec(memory_space=pltpu.VMEM),
        scratch_shapes=[pltpu.SemaphoreType.DMA, pltpu.SemaphoreType.DMA],
        compiler_params=pltpu.CompilerParams(collective_id=0),
    )(rt_pad)


def _mega_body(
    xbf_ref, gwm_ref, gws_ref, w1_ref, w2_ref, out_ref,
    xloc_ref, gwloc_ref, cacc_ref, csend_ref, zrecv_ref, sbf_ref, brecv_ref,
    send_sems, recv_sems,
):
    e = pl.program_id(0)
    h = pl.program_id(1)
    my_x = lax.axis_index("x")
    my_y = lax.axis_index("y")
    my_z = lax.axis_index("z")
    xy = 2 * my_x + my_y
    o = TC * xy
    zp = (my_x, my_y, 1 - my_z)
    xn = (1 - my_x, my_y, my_z)
    yn = (my_x, 1 - my_y, my_z)
    dg = (1 - my_x, 1 - my_y, my_z)

    def disp_x():
        return pltpu.make_async_remote_copy(
            src_ref=xbf_ref.at[pl.ds(o, TC), :],
            dst_ref=xloc_ref.at[1],
            send_sem=send_sems.at[0],
            recv_sem=recv_sems.at[0],
            device_id=zp,
            device_id_type=MESH,
        )

    def disp_gw():
        return pltpu.make_async_remote_copy(
            src_ref=gws_ref,
            dst_ref=gwloc_ref.at[1],
            send_sem=send_sems.at[1],
            recv_sem=recv_sems.at[1],
            device_id=zp,
            device_id_type=MESH,
        )

    def zsum():
        return pltpu.make_async_remote_copy(
            src_ref=csend_ref,
            dst_ref=zrecv_ref,
            send_sem=send_sems.at[2],
            recv_sem=recv_sems.at[2],
            device_id=zp,
            device_id_type=MESH,
        )

    def bcast(slot, dev, dst_slot):
        return pltpu.make_async_remote_copy(
            src_ref=sbf_ref,
            dst_ref=brecv_ref.at[dst_slot],
            send_sem=send_sems.at[slot],
            recv_sem=recv_sems.at[slot],
            device_id=dev,
            device_id_type=MESH,
        )

    @pl.when(jnp.logical_and(e == 0, h == 0))
    def _():
        barrier = pltpu.get_barrier_semaphore()
        for dev in (zp, xn, yn, dg):
            pl.semaphore_signal(barrier, inc=1, device_id=dev,
                                device_id_type=MESH)
        pl.semaphore_wait(barrier, 4)
        xloc_ref[0] = xbf_ref[pl.ds(o, TC), :]
        gwloc_ref[0] = gwm_ref[:, pl.ds(o, TC), :]
        disp_x().start()
        disp_gw().start()

    @pl.when(jnp.logical_and(e == 0, h == 1))
    def _():
        disp_x().wait_recv()
        disp_gw().wait_recv()

    hid = jnp.maximum(
        jnp.dot(xloc_ref[h], w1_ref[0].astype(jnp.bfloat16),
                preferred_element_type=jnp.float32),
        0.0,
    )
    hid = (hid * gwloc_ref[h, e]).astype(jnp.bfloat16)
    contrib = jnp.dot(hid, w2_ref[0].astype(jnp.bfloat16),
                      preferred_element_type=jnp.float32)

    @pl.when(e == 0)
    def _():
        cacc_ref[h] = contrib

    @pl.when(e != 0)
    def _():
        cacc_ref[h] = cacc_ref[h] + contrib

    @pl.when(jnp.logical_and(e == EL - 1, h == 1))
    def _():
        csend_ref[...] = cacc_ref[1].astype(jnp.bfloat16)
        zs = zsum()
        zs.start()
        zs.wait_recv()
        s = cacc_ref[0] + zrecv_ref[...].astype(jnp.float32)

        sbf_ref[...] = s.astype(jnp.bfloat16)
        xy_xn = 2 * (1 - my_x) + my_y
        xy_yn = 2 * my_x + (1 - my_y)
        xy_dg = 2 * (1 - my_x) + (1 - my_y)
        bx = bcast(3, xn, xy)
        by = bcast(4, yn, xy)
        bd = bcast(5, dg, xy)
        bx.start()
        by.start()
        bd.start()
        bcast(3, xn, xy_xn).wait_recv()
        bcast(4, yn, xy_yn).wait_recv()
        bcast(5, dg, xy_dg).wait_recv()

        for j in range(4):
            out_ref[j * TC : (j + 1) * TC, :] = jnp.where(
                xy == j, s, brecv_ref[j].astype(jnp.float32)
            )

        disp_x().wait_send()
        disp_gw().wait_send()
        zs.wait_send()
        bx.wait_send()
        by.wait_send()
        bd.wait_send()


def _mega(x_bf, gw_mine, gw_send, W1, W2):
    return pl.pallas_call(
        _mega_body,
        grid=(EL, 2),
        out_shape=jax.ShapeDtypeStruct((TL, D), jnp.float32),
        in_specs=[
            pl.BlockSpec(memory_space=pltpu.VMEM),
            pl.BlockSpec(memory_space=pltpu.VMEM),
            pl.BlockSpec(memory_space=pltpu.VMEM),
            pl.BlockSpec((1, D, F), lambda e, h: (e, 0, 0)),
            pl.BlockSpec((1, F, D), lambda e, h: (e, 0, 0)),
        ],
        out_specs=pl.BlockSpec((TL, D), lambda e, h: (0, 0)),
        scratch_shapes=[
            pltpu.VMEM((2, TC, D), jnp.bfloat16),
            pltpu.VMEM((2, EL, TC, 1), jnp.float32),
            pltpu.VMEM((2, TC, D), jnp.float32),
            pltpu.VMEM((TC, D), jnp.bfloat16),
            pltpu.VMEM((TC, D), jnp.bfloat16),
            pltpu.VMEM((TC, D), jnp.bfloat16),
            pltpu.VMEM((4, TC, D), jnp.bfloat16),
            pltpu.SemaphoreType.DMA((6,)),
            pltpu.SemaphoreType.DMA((6,)),
        ],
        compiler_params=pltpu.CompilerParams(
            collective_id=1, vmem_limit_bytes=64 * 1024 * 1024
        ),
    )(x_bf, gw_mine, gw_send, W1, W2)


def kernel(x, router, W1, W2):
    my_z = lax.axis_index("z")
    xy = 2 * lax.axis_index("x") + lax.axis_index("y")
    o = TC * xy

    rt_pad = jnp.zeros((8, D), jnp.float32).at[:EL].set(router.T)
    rtf_pad = _router_exchange(rt_pad)
    router_t_full = jnp.concatenate([rtf_pad[:EL], rtf_pad[8 : 8 + EL]], axis=0)

    gates = jnp.dot(x, router_t_full.T, precision=lax.Precision.HIGHEST)
    i1 = jnp.argmax(gates, axis=1)
    v1 = jnp.max(gates, axis=1)
    masked = gates - jnp.where(
        jax.nn.one_hot(i1, E, dtype=jnp.float32) > 0, jnp.inf, 0.0
    )
    i2 = jnp.argmax(masked, axis=1)
    v2 = jnp.max(masked, axis=1)
    e2 = jnp.exp(v2 - v1)
    w1g = 1.0 / (1.0 + e2)
    w2g = e2 / (1.0 + e2)
    gw = (
        w1g[:, None] * jax.nn.one_hot(i1, E, dtype=jnp.float32)
        + w2g[:, None] * jax.nn.one_hot(i2, E, dtype=jnp.float32)
    )
    gw_mine = lax.dynamic_slice(gw, (0, my_z * EL), (TL, EL)).T[:, :, None]
    gw_send = lax.dynamic_slice(
        gw, (o, (1 - my_z) * EL), (TC, EL)
    ).T[:, :, None]

    return _mega(x.astype(jnp.bfloat16), gw_mine, gw_send, W1, W2)


# device time: 58391 ns/iter; 1.9296x vs baseline; 1.1291x over previous
import jax
import jax.numpy as jnp
from jax import lax
from jax.experimental import pallas as pl
from jax.experimental.pallas import tpu as pltpu

T = 1024
TL = 512
TC = 128
D = 1024
F = 2048
E = 8
EL = 4

MESH = pl.DeviceIdType.MESH

S_DX = 0
S_DG = 1
S_ZS = 2
S_BX = 6
S_BY = 7
S_BD = 8
S_RT = 9
NSEM = 10


FC = 1024
NF = F // FC


def _body(
    x_ref, rt_ref, w1_ref, w2_ref, out_ref,
    xloc_ref, rtf_ref, gw16_ref, gwcat_ref, gwsend_ref, w1b_ref, w2b_ref,
    pacc_ref, cacc_ref,
    csend_ref, zrecv_ref, sbf_ref, brecv_ref,
    send_sems, recv_sems,
):
    e = pl.program_id(0)
    f = pl.program_id(1)
    my_x = lax.axis_index("x")
    my_y = lax.axis_index("y")
    my_z = lax.axis_index("z")
    xy = 2 * my_x + my_y
    o = TC * xy
    zp = (my_x, my_y, 1 - my_z)
    xn = (1 - my_x, my_y, my_z)
    yn = (my_x, 1 - my_y, my_z)
    dg = (1 - my_x, 1 - my_y, my_z)

    def rt_x():
        return pltpu.make_async_remote_copy(
            src_ref=rt_ref,
            dst_ref=rtf_ref.at[pl.ds(my_z * 8, 8), :],
            send_sem=send_sems.at[S_RT],
            recv_sem=recv_sems.at[S_RT],
            device_id=zp,
            device_id_type=MESH,
        )

    def disp_x():
        return pltpu.make_async_remote_copy(
            src_ref=xloc_ref.at[0],
            dst_ref=xloc_ref.at[1],
            send_sem=send_sems.at[S_DX],
            recv_sem=recv_sems.at[S_DX],
            device_id=zp,
            device_id_type=MESH,
        )

    def disp_gw():
        return pltpu.make_async_remote_copy(
            src_ref=gwsend_ref,
            dst_ref=gwcat_ref.at[pl.ds(TC, TC), :],
            send_sem=send_sems.at[S_DG],
            recv_sem=recv_sems.at[S_DG],
            device_id=zp,
            device_id_type=MESH,
        )

    def zsum(ei):
        return pltpu.make_async_remote_copy(
            src_ref=csend_ref.at[ei],
            dst_ref=zrecv_ref.at[ei],
            send_sem=send_sems.at[S_ZS + ei],
            recv_sem=recv_sems.at[S_ZS + ei],
            device_id=zp,
            device_id_type=MESH,
        )

    def bcast(slot, dev, dst_slot):
        return pltpu.make_async_remote_copy(
            src_ref=sbf_ref,
            dst_ref=brecv_ref.at[dst_slot],
            send_sem=send_sems.at[slot],
            recv_sem=recv_sems.at[slot],
            device_id=dev,
            device_id_type=MESH,
        )

    @pl.when(jnp.logical_and(e == 0, f == 0))
    def _():
        barrier = pltpu.get_barrier_semaphore()
        for dev in (zp, xn, yn, dg):
            pl.semaphore_signal(barrier, inc=1, device_id=dev,
                                device_id_type=MESH)
        pl.semaphore_wait(barrier, 4)

        xloc_ref[0] = x_ref[pl.ds(o, TC), :].astype(jnp.bfloat16)
        disp_x().start()

        rtf_ref[pl.ds(my_z * 8, 8), :] = rt_ref[...]
        rt = rt_x()
        rt.start()
        rt.wait_recv()
        gates = lax.dot_general(
            x_ref[...], rtf_ref[...],
            (((1,), (1,)), ((), ())),
            precision=lax.Precision.HIGHEST,
            preferred_element_type=jnp.float32,
        )
        col = lax.broadcasted_iota(jnp.int32, (TL, 16), 1)
        valid = (col % 8) < EL
        g = jnp.where(valid, gates, -1e30)
        v1 = jnp.max(g, axis=1, keepdims=True)
        m1 = (g == v1).astype(jnp.float32)
        g2 = jnp.where(g == v1, -1e30, g)
        v2 = jnp.max(g2, axis=1, keepdims=True)
        m2 = (g2 == v2).astype(jnp.float32)
        e2 = jnp.exp(v2 - v1)
        w1g = 1.0 / (1.0 + e2)
        gw16 = w1g * m1 + (e2 * w1g) * m2

        i16 = lax.broadcasted_iota(jnp.int32, (16, EL), 0)
        j4 = lax.broadcasted_iota(jnp.int32, (16, EL), 1)
        sel_mine = (i16 == 8 * my_z + j4).astype(jnp.float32)
        sel_send = (i16 == 8 * (1 - my_z) + j4).astype(jnp.float32)
        gw16_ref[...] = gw16
        rows = gw16_ref[pl.ds(o, TC), :]
        gwcat_ref[pl.ds(0, TC), :] = jnp.dot(
            rows, sel_mine, preferred_element_type=jnp.float32
        )
        gwsend_ref[...] = jnp.dot(
            rows, sel_send, preferred_element_type=jnp.float32
        )
        disp_gw().start()

    w1b_ref[...] = w1_ref[0].astype(jnp.bfloat16)
    w2b_ref[...] = w2_ref[0].astype(jnp.bfloat16)

    @pl.when(jnp.logical_and(e == 0, f == 0))
    def _():
        disp_x().wait_recv()
        disp_gw().wait_recv()

    xv = jnp.reshape(xloc_ref[...], (2 * TC, D))
    onehot_e = (lax.broadcasted_iota(jnp.int32, (EL, 1), 0) == e).astype(
        jnp.float32
    )
    gwv = jnp.dot(gwcat_ref[...], onehot_e,
                  preferred_element_type=jnp.float32)
    hid = jnp.maximum(
        jnp.dot(xv, w1b_ref[...], preferred_element_type=jnp.float32), 0.0
    )
    hid = (hid * gwv).astype(jnp.bfloat16)
    contrib = jnp.dot(hid, w2b_ref[...], preferred_element_type=jnp.float32)

    @pl.when(f == 0)
    def _():
        pacc_ref[...] = contrib

    @pl.when(f != 0)
    def _():
        pacc_ref[...] = pacc_ref[...] + contrib

    @pl.when(jnp.logical_and(f == NF - 1, e == 0))
    def _():
        cacc_ref[...] = pacc_ref[pl.ds(0, TC), :]

    @pl.when(jnp.logical_and(f == NF - 1, e != 0))
    def _():
        cacc_ref[...] = cacc_ref[...] + pacc_ref[pl.ds(0, TC), :]

    for ei in range(EL):
        @pl.when(jnp.logical_and(f == NF - 1, e == ei))
        def _(ei=ei):
            csend_ref[ei] = pacc_ref[pl.ds(TC, TC), :].astype(jnp.bfloat16)
            zsum(ei).start()

    @pl.when(jnp.logical_and(e == EL - 1, f == NF - 1))
    def _():
        for ei in range(EL):
            zsum(ei).wait_recv()
        s = cacc_ref[...] + (
            zrecv_ref[0].astype(jnp.float32)
            + zrecv_ref[1].astype(jnp.float32)
            + zrecv_ref[2].astype(jnp.float32)
            + zrecv_ref[3].astype(jnp.float32)
        )

        sbf_ref[...] = s.astype(jnp.bfloat16)
        xy_xn = 2 * (1 - my_x) + my_y
        xy_yn = 2 * my_x + (1 - my_y)
        xy_dg = 2 * (1 - my_x) + (1 - my_y)
        bx = bcast(S_BX, xn, xy)
        by = bcast(S_BY, yn, xy)
        bd = bcast(S_BD, dg, xy)
        bx.start()
        by.start()
        bd.start()

        out_ref[pl.ds(o, TC), :] = s
        bcast(S_BX, xn, xy_xn).wait_recv()
        out_ref[pl.ds(TC * xy_xn, TC), :] = brecv_ref[xy_xn].astype(jnp.float32)
        bcast(S_BY, yn, xy_yn).wait_recv()
        out_ref[pl.ds(TC * xy_yn, TC), :] = brecv_ref[xy_yn].astype(jnp.float32)
        bcast(S_BD, dg, xy_dg).wait_recv()
        out_ref[pl.ds(TC * xy_dg, TC), :] = brecv_ref[xy_dg].astype(jnp.float32)

        rt_x().wait_send()
        disp_x().wait_send()
        disp_gw().wait_send()
        for ei in range(EL):
            zsum(ei).wait_send()
        bx.wait_send()
        by.wait_send()
        bd.wait_send()


def kernel(x, router, W1, W2):
    rt_pad = jnp.zeros((8, D), jnp.float32).at[:EL].set(router.T)
    return pl.pallas_call(
        _body,
        grid=(EL, NF),
        out_shape=jax.ShapeDtypeStruct((TL, D), jnp.float32),
        in_specs=[
            pl.BlockSpec(memory_space=pltpu.VMEM),
            pl.BlockSpec(memory_space=pltpu.VMEM),
            pl.BlockSpec((1, D, FC), lambda e, f: (e, 0, f)),
            pl.BlockSpec((1, FC, D), lambda e, f: (e, f, 0)),
        ],
        out_specs=pl.BlockSpec((TL, D), lambda e, f: (0, 0)),
        scratch_shapes=[
            pltpu.VMEM((2, TC, D), jnp.bfloat16),
            pltpu.VMEM((16, D), jnp.float32),
            pltpu.VMEM((TL, 16), jnp.float32),
            pltpu.VMEM((2 * TC, EL), jnp.float32),
            pltpu.VMEM((TC, EL), jnp.float32),
            pltpu.VMEM((D, FC), jnp.bfloat16),
            pltpu.VMEM((FC, D), jnp.bfloat16),
            pltpu.VMEM((2 * TC, D), jnp.float32),
            pltpu.VMEM((TC, D), jnp.float32),
            pltpu.VMEM((EL, TC, D), jnp.bfloat16),
            pltpu.VMEM((EL, TC, D), jnp.bfloat16),
            pltpu.VMEM((TC, D), jnp.bfloat16),
            pltpu.VMEM((4, TC, D), jnp.bfloat16),
            pltpu.SemaphoreType.DMA((NSEM,)),
            pltpu.SemaphoreType.DMA((NSEM,)),
        ],
        compiler_params=pltpu.CompilerParams(
            collective_id=0, vmem_limit_bytes=64 * 1024 * 1024
        ),
    )(x, rt_pad, W1, W2)
